# baseline (speedup 1.0000x reference)
import functools

import jax
import jax.numpy as jnp
from jax import lax
from jax.experimental import pallas as pl
from jax.experimental.pallas import tpu as pltpu
from jax.experimental.pallas import tpu_sc as plsc

_PAD_D = 8
_NUM_WORKERS = 32
_IDX_CHUNK = 128
_ROW_BLOCK = 1024


def _full16(v):
    return jnp.full((16,), v, jnp.int32)


def _sc_gather_components(table, idx_t):
    e_total = idx_t.shape[1]
    e_per_w = e_total // _NUM_WORKERS
    rows_per_w = 4 * e_per_w
    chunks = e_per_w // _IDX_CHUNK
    lane_grp = e_per_w // 128
    mesh = plsc.VectorSubcoreMesh(core_axis_name="c", subcore_axis_name="s")

    @functools.partial(
        pl.kernel,
        mesh=mesh,
        out_type=jax.ShapeDtypeStruct((12, e_total // 128, 128), jnp.float32),
        compiler_params=pltpu.CompilerParams(
            use_tc_tiling_on_sc=False, needs_layout_passes=False),
        scratch_types=[
            pltpu.VMEM((4, e_per_w), jnp.int32),
            pltpu.VMEM((rows_per_w, _PAD_D), jnp.float32),
            pltpu.VMEM((12, lane_grp, 128), jnp.float32),
            pltpu.SemaphoreType.DMA,
        ],
    )
    def gather_kernel(table_hbm, idx_hbm, out_hbm, idx_v, rows_v, comp_v, sem):
        wid = lax.axis_index("s") * 2 + lax.axis_index("c")
        base_e = wid * e_per_w
        for f in range(4):
            pltpu.sync_copy(idx_hbm.at[f, pl.ds(base_e, e_per_w)],
                            idx_v.at[f])
        copies = []
        for f in range(4):
            for c in range(4):
                copies.append(
                    pltpu.async_copy(
                        table_hbm.at[idx_v.at[f, pl.ds(c * _IDX_CHUNK,
                                                       _IDX_CHUNK)]],
                        rows_v.at[pl.ds((f * chunks + c) * _IDX_CHUNK,
                                        _IDX_CHUNK)],
                        sem,
                    )
                )
        for cp in copies:
            cp.wait()

        iot = lax.iota(jnp.int32, 16)
        for j4 in range(lane_grp):
            @pl.loop(0, 128, step=16)
            def _(m, j4=j4):
                g = j4 * 128 + m
                for f in range(4):
                    ridx = iot + (f * e_per_w + g)
                    for t in range(3):
                        comp_v[f * 3 + t, j4, pl.ds(m, 16)] = plsc.load_gather(
                            rows_v, [ridx, _full16(t)])

        pltpu.sync_copy(
            comp_v, out_hbm.at[:, pl.ds(wid * lane_grp, lane_grp), :])

    return gather_kernel(table, idx_t)


def _elastic_body(c_ref, p_ref, meas_ref, lam_ref, mu_ref, out_ref):
    a = [c_ref[i] for i in range(12)]
    b = [p_ref[i] for i in range(16)]
    f_mat = [[None] * 3 for _ in range(3)]
    for t in range(3):
        for l in range(3):
            acc = a[0 * 3 + t] * b[0 * 4 + l]
            for f in range(1, 4):
                acc += a[f * 3 + t] * b[f * 4 + l]
            f_mat[t][l] = acc
    ic = f_mat[0][0] * f_mat[0][0]
    for t in range(3):
        for l in range(3):
            if t or l:
                ic += f_mat[t][l] * f_mat[t][l]
    det = (
        f_mat[0][0] * (f_mat[1][1] * f_mat[2][2] - f_mat[1][2] * f_mat[2][1])
        - f_mat[0][1] * (f_mat[1][0] * f_mat[2][2] - f_mat[1][2] * f_mat[2][0])
        + f_mat[0][2] * (f_mat[1][0] * f_mat[2][1] - f_mat[1][1] * f_mat[2][0])
    )
    meas, lam_v, mu_v = meas_ref[...], lam_ref[...], mu_ref[...]
    alpha = 0.75 * mu_v / lam_v + 1.0
    ic_ver = jnp.maximum(ic + 1.0, 0.0) + 1e-30
    dens = (
        0.5 * mu_v * (ic - 3.0)
        + 0.5 * lam_v * (det - alpha) ** 2
        - 0.5 * mu_v * jnp.log(ic_ver)
    )
    out_ref[0, 0] = jnp.sum(dens * meas)


def _kinetic_body(m_ref, dt_ref, dn_ref, out_ref):
    i = pl.program_id(0)

    @pl.when(i == 0)
    def _():
        out_ref[0, 0] = 0.0

    m = m_ref[...]
    acc = jnp.float32(0.0)
    for k in range(3):
        s = jnp.sum(m * dt_ref[k : k + 1, :], axis=1, keepdims=True)
        acc += jnp.sum(s * dn_ref[:, k : k + 1])
    out_ref[0, 0] += acc


def kernel(position, time_step, state_position, velocity,
           external_acceleration, int_density_matrix, elements, polynomials,
           measure, lam, mu):
    n = position.shape[0]
    e = elements.shape[0]
    eb = e // 128
    f32 = jnp.float32
    dt = jnp.asarray(time_step, f32)
    coeff = 0.5 / (dt * dt)

    table = jnp.pad(position, ((0, 0), (0, _PAD_D - 3)))
    idx_t = elements.T.astype(jnp.int32)
    comp = _sc_gather_components(table, idx_t)

    polyc = jnp.transpose(polynomials, (1, 2, 0)).reshape(16, eb, 128)

    elastic = pl.pallas_call(
        _elastic_body,
        out_shape=jax.ShapeDtypeStruct((1, 1), f32),
        in_specs=[
            pl.BlockSpec((12, eb, 128), lambda: (0, 0, 0)),
            pl.BlockSpec((16, eb, 128), lambda: (0, 0, 0)),
            pl.BlockSpec((eb, 128), lambda: (0, 0)),
            pl.BlockSpec((eb, 128), lambda: (0, 0)),
            pl.BlockSpec((eb, 128), lambda: (0, 0)),
        ],
        out_specs=pl.BlockSpec(memory_space=pltpu.SMEM),
    )(comp, polyc, measure.reshape(eb, 128),
      lam.reshape(eb, 128), mu.reshape(eb, 128))[0, 0]

    delta_t = (position.T - state_position.T - velocity.T * dt
               - external_acceleration.T * (dt * dt)).astype(f32)
    dt_t = jnp.zeros((8, n), f32).at[:3, :].set(delta_t)
    dn = jnp.zeros((n, 8), f32).at[:, :3].set(delta_t.T)

    kin_raw = pl.pallas_call(
        _kinetic_body,
        grid=(n // _ROW_BLOCK,),
        out_shape=jax.ShapeDtypeStruct((1, 1), f32),
        in_specs=[
            pl.BlockSpec((_ROW_BLOCK, n), lambda i: (i, 0)),
            pl.BlockSpec((8, n), lambda i: (0, 0)),
            pl.BlockSpec((_ROW_BLOCK, 8), lambda i: (i, 0)),
        ],
        out_specs=pl.BlockSpec(memory_space=pltpu.SMEM),
    )(int_density_matrix, dt_t, dn)[0, 0]

    return (coeff * kin_raw + elastic).astype(f32)


# speedup vs baseline: 1.0837x; 1.0837x over previous
import functools

import jax
import jax.numpy as jnp
from jax import lax
from jax.experimental import pallas as pl
from jax.experimental.pallas import tpu as pltpu
from jax.experimental.pallas import tpu_sc as plsc

_PAD_D = 8
_NUM_WORKERS = 32
_IDX_CHUNK = 128
_ROW_BLOCK = 512


def _full16(v):
    return jnp.full((16,), v, jnp.int32)


def _sc_gather_components(table, idx_t):
    e_total = idx_t.shape[1]
    e_per_w = e_total // _NUM_WORKERS
    rows_per_w = 4 * e_per_w
    chunks = e_per_w // _IDX_CHUNK
    lane_grp = e_per_w // 128
    mesh = plsc.VectorSubcoreMesh(core_axis_name="c", subcore_axis_name="s")

    @functools.partial(
        pl.kernel,
        mesh=mesh,
        out_type=jax.ShapeDtypeStruct((12, e_total // 128, 128), jnp.float32),
        compiler_params=pltpu.CompilerParams(
            use_tc_tiling_on_sc=False, needs_layout_passes=False),
        scratch_types=[
            pltpu.VMEM((4, e_per_w), jnp.int32),
            pltpu.VMEM((rows_per_w, _PAD_D), jnp.float32),
            pltpu.VMEM((12, lane_grp, 128), jnp.float32),
            pltpu.SemaphoreType.DMA,
        ],
    )
    def gather_kernel(table_hbm, idx_hbm, out_hbm, idx_v, rows_v, comp_v, sem):
        wid = lax.axis_index("s") * 2 + lax.axis_index("c")
        base_e = wid * e_per_w
        for f in range(4):
            pltpu.sync_copy(idx_hbm.at[f, pl.ds(base_e, e_per_w)],
                            idx_v.at[f])
        copies = []
        for f in range(4):
            for c in range(4):
                copies.append(
                    pltpu.async_copy(
                        table_hbm.at[idx_v.at[f, pl.ds(c * _IDX_CHUNK,
                                                       _IDX_CHUNK)]],
                        rows_v.at[pl.ds((f * chunks + c) * _IDX_CHUNK,
                                        _IDX_CHUNK)],
                        sem,
                    )
                )
        for cp in copies:
            cp.wait()

        iot = lax.iota(jnp.int32, 16)
        for j4 in range(lane_grp):
            @pl.loop(0, 128, step=16)
            def _(m, j4=j4):
                g = j4 * 128 + m
                for f in range(4):
                    ridx = iot + (f * e_per_w + g)
                    for t in range(3):
                        comp_v[f * 3 + t, j4, pl.ds(m, 16)] = plsc.load_gather(
                            rows_v, [ridx, _full16(t)])

        pltpu.sync_copy(
            comp_v, out_hbm.at[:, pl.ds(wid * lane_grp, lane_grp), :])

    return gather_kernel(table, idx_t)


def _elastic_sum(c_ref, p_ref, meas_ref, lam_ref, mu_ref):
    a = [c_ref[i] for i in range(12)]
    b = [p_ref[i] for i in range(16)]
    f_mat = [[None] * 3 for _ in range(3)]
    for t in range(3):
        for l in range(3):
            acc = a[0 * 3 + t] * b[0 * 4 + l]
            for f in range(1, 4):
                acc += a[f * 3 + t] * b[f * 4 + l]
            f_mat[t][l] = acc
    ic = f_mat[0][0] * f_mat[0][0]
    for t in range(3):
        for l in range(3):
            if t or l:
                ic += f_mat[t][l] * f_mat[t][l]
    det = (
        f_mat[0][0] * (f_mat[1][1] * f_mat[2][2] - f_mat[1][2] * f_mat[2][1])
        - f_mat[0][1] * (f_mat[1][0] * f_mat[2][2] - f_mat[1][2] * f_mat[2][0])
        + f_mat[0][2] * (f_mat[1][0] * f_mat[2][1] - f_mat[1][1] * f_mat[2][0])
    )
    meas, lam_v, mu_v = meas_ref[...], lam_ref[...], mu_ref[...]
    alpha = 0.75 * mu_v / lam_v + 1.0
    ic_ver = jnp.maximum(ic + 1.0, 0.0) + 1e-30
    dens = (
        0.5 * mu_v * (ic - 3.0)
        + 0.5 * lam_v * (det - alpha) ** 2
        - 0.5 * mu_v * jnp.log(ic_ver)
    )
    return jnp.sum(dens * meas)


def _fused_body(m_ref, dt_ref, dn_ref, comp_ref, poly_ref, meas_ref,
                lam_ref, mu_ref, coeff_ref, out_ref):
    i = pl.program_id(0)

    @pl.when(i == 0)
    def _():
        out_ref[0, 0] = 0.0

    m = m_ref[...]
    acc = jnp.float32(0.0)
    for k in range(3):
        s = jnp.sum(m * dt_ref[k : k + 1, :], axis=1, keepdims=True)
        acc += jnp.sum(s * dn_ref[:, k : k + 1])
    out_ref[0, 0] += acc * coeff_ref[0, 0]

    @pl.when(i == pl.num_programs(0) - 1)
    def _():
        out_ref[0, 0] += _elastic_sum(comp_ref, poly_ref, meas_ref,
                                      lam_ref, mu_ref)


def kernel(position, time_step, state_position, velocity,
           external_acceleration, int_density_matrix, elements, polynomials,
           measure, lam, mu):
    n = position.shape[0]
    e = elements.shape[0]
    eb = e // 128
    f32 = jnp.float32
    dt = jnp.asarray(time_step, f32)
    coeff = (0.5 / (dt * dt)).astype(f32).reshape(1, 1)

    table = jnp.pad(position, ((0, 0), (0, _PAD_D - 3)))
    idx_t = elements.T.astype(jnp.int32)
    comp = _sc_gather_components(table, idx_t)

    polyc = jnp.transpose(polynomials, (1, 2, 0)).reshape(16, eb, 128)

    delta_t = (position.T - state_position.T - velocity.T * dt
               - external_acceleration.T * (dt * dt)).astype(f32)
    dt_t = jnp.zeros((8, n), f32).at[:3, :].set(delta_t)
    dn = jnp.zeros((n, 8), f32).at[:, :3].set(delta_t.T)

    total = pl.pallas_call(
        _fused_body,
        grid=(n // _ROW_BLOCK,),
        out_shape=jax.ShapeDtypeStruct((1, 1), f32),
        in_specs=[
            pl.BlockSpec((_ROW_BLOCK, n), lambda i: (i, 0)),
            pl.BlockSpec((8, n), lambda i: (0, 0)),
            pl.BlockSpec((_ROW_BLOCK, 8), lambda i: (i, 0)),
            pl.BlockSpec((12, eb, 128), lambda i: (0, 0, 0)),
            pl.BlockSpec((16, eb, 128), lambda i: (0, 0, 0)),
            pl.BlockSpec((eb, 128), lambda i: (0, 0)),
            pl.BlockSpec((eb, 128), lambda i: (0, 0)),
            pl.BlockSpec((eb, 128), lambda i: (0, 0)),
            pl.BlockSpec(memory_space=pltpu.SMEM),
        ],
        out_specs=pl.BlockSpec(memory_space=pltpu.SMEM),
    )(int_density_matrix, dt_t, dn, comp, polyc,
      measure.reshape(eb, 128), lam.reshape(eb, 128), mu.reshape(eb, 128),
      coeff)[0, 0]

    return total.astype(f32)
